# baseline (speedup 1.0000x reference)
import functools

import jax
import jax.numpy as jnp
from jax import lax
from jax.experimental import pallas as pl
from jax.experimental.pallas import tpu as pltpu
from jax.experimental.pallas import tpu_sc as plsc

_RB = 512


def _tree_norm(x):
    t = x * x
    c = t.shape[0]
    while c > 1:
        h = c // 2
        t = t[:h] + t[h:c]
        c = h
    return jnp.sqrt(t) + 1e-12


def _corr_body(qt_ref, kt_ref, o_ref):
    n = kt_ref.shape[2]
    rb = qt_ref.shape[2]
    qt = qt_ref[0]
    kt = kt_ref[0]
    nq = _tree_norm(qt)
    nk = _tree_norm(kt)
    s0 = lax.dot_general(
        (qt / nq).astype(jnp.bfloat16), (kt / nk).astype(jnp.bfloat16),
        (((0,), (0,)), ((), ())),
        preferred_element_type=jnp.float32,
    )
    ch = 128
    neg = jnp.full((rb, ch), -jnp.inf, jnp.float32)
    v1, v2, v3 = neg, neg, neg
    for c in range(n // ch):
        x = s0[:, c * ch:(c + 1) * ch]
        t1 = jnp.maximum(v1, x)
        l1 = jnp.minimum(v1, x)
        t2 = jnp.maximum(v2, l1)
        l2 = jnp.minimum(v2, l1)
        v3 = jnp.maximum(v3, l2)
        v1, v2 = t1, t2
    width = ch
    while width > 1:
        h = width // 2
        a1, b1 = v1[:, :h], v1[:, h:width]
        a2, b2 = v2[:, :h], v2[:, h:width]
        a3, b3 = v3[:, :h], v3[:, h:width]
        for x in (b1, b2, b3):
            t1 = jnp.maximum(a1, x)
            l1 = jnp.minimum(a1, x)
            t2 = jnp.maximum(a2, l1)
            l2 = jnp.minimum(a2, l1)
            a3 = jnp.maximum(a3, l2)
            a1, a2 = t1, t2
        v1, v2, v3 = a1, a2, a3
        width = h
    m3 = v3
    ci = lax.broadcasted_iota(jnp.int32, (1, n), 1)
    rhs_t = jnp.concatenate(
        [nk * (ci >> 6).astype(jnp.float32),
         nk * (ci & 63).astype(jnp.float32),
         nk], axis=0)
    wm = jnp.where(s0 >= m3, s0, 0.0)
    nums = lax.dot_general(
        wm, rhs_t, (((1,), (1,)), ((), ())),
        preferred_element_type=jnp.float32,
    )
    den = nums[:, 2:3]
    o_ref[0] = jnp.concatenate([nums[:, 0:1] / den, nums[:, 1:2] / den], axis=1)


def _pred_corr(fx, fy):
    B, C, N = fx.shape
    return pl.pallas_call(
        _corr_body,
        grid=(B, N // _RB),
        in_specs=[
            pl.BlockSpec((1, C, _RB), lambda b, i: (b, 0, i)),
            pl.BlockSpec((1, C, N), lambda b, i: (b, 0, 0)),
        ],
        out_specs=pl.BlockSpec((1, _RB, 2), lambda b, i: (b, i, 0)),
        out_shape=jax.ShapeDtypeStruct((B, N, 2), jnp.float32),
    )(fx, fy)


def _sc_gather(table, src_flat, n_tab, nt, h, scale):
    p = (((nt + 31) // 32) + 3) // 4 * 4
    rem = (nt - 30 * p) // 2
    n_iter = (p + 15) // 16
    buf = 16 * n_iter * 2
    half = nt // 2
    mesh = plsc.VectorSubcoreMesh(core_axis_name="c", subcore_axis_name="s")

    @functools.partial(
        pl.kernel,
        mesh=mesh,
        compiler_params=pltpu.CompilerParams(needs_layout_passes=False),
        out_type=jax.ShapeDtypeStruct((2 * nt,), jnp.float32),
        scratch_types=[
            pltpu.VMEM((n_tab,), jnp.float32),
            pltpu.VMEM((buf,), jnp.float32),
            pltpu.VMEM((buf,), jnp.float32),
        ],
    )
    def k(table_hbm, src_hbm, out_hbm, tab_v, src_v, out_v):
        wid = lax.axis_index("s") * 2 + lax.axis_index("c")
        pltpu.sync_copy(table_hbm, tab_v)

        @pl.when(wid < 30)
        def _():
            pltpu.sync_copy(src_hbm.at[pl.ds(wid * 2 * p, 2 * p)],
                            src_v.at[pl.ds(0, 2 * p)])

        @pl.when(wid >= 30)
        def _():
            pltpu.sync_copy(
                src_hbm.at[pl.ds(60 * p + (wid - 30) * 2 * rem, 2 * rem)],
                src_v.at[pl.ds(0, 2 * rem)])

        tbase = jnp.where(wid < 30, wid * p, 30 * p + (wid - 30) * rem)
        lane = lax.iota(jnp.int32, 16)
        fh = float(h)
        for j in range(n_iter):
            tl = j * 16 + lane
            s_h = plsc.load_gather(src_v, [2 * tl])
            s_w = plsc.load_gather(src_v, [2 * tl + 1])

            def rhe(y):
                y = y * fh
                i = (y + 0.5).astype(jnp.int32)
                is_half = (i.astype(jnp.float32) - y) == 0.5
                i = i - jnp.where(is_half, i & 1, 0)
                return jnp.minimum(jnp.maximum(i, 0), h - 1)

            i0 = rhe(s_h)
            i1 = rhe(s_w)
            b_off = jnp.where(tbase + tl >= half, n_tab // 2, 0)
            idx = b_off + (i0 * h + i1) * 2
            g0 = plsc.load_gather(tab_v, [idx])
            g1 = plsc.load_gather(tab_v, [idx + 1])
            pos = j * 32 + 2 * lane
            plsc.store_scatter(out_v, [pos], g0 * scale)
            plsc.store_scatter(out_v, [pos + 1], g1 * scale)

        @pl.when(wid < 30)
        def _():
            pltpu.sync_copy(out_v.at[pl.ds(0, 2 * p)],
                            out_hbm.at[pl.ds(wid * 2 * p, 2 * p)])

        @pl.when(wid >= 30)
        def _():
            pltpu.sync_copy(
                out_v.at[pl.ds(0, 2 * rem)],
                out_hbm.at[pl.ds(60 * p + (wid - 30) * 2 * rem, 2 * rem)])

    return k(table, src_flat)


def kernel(img1_feature, img2_feature, mask_1, mask_2, slic, src_pixel_group):
    B, C, H, W = img1_feature.shape
    N = H * W
    fy = img2_feature.reshape(B, C, N)
    fx = img1_feature.reshape(B, C, N)
    pred = _pred_corr(fx, fy)
    pred_corr = pred.reshape(B, H, W, 2)

    _, S, G, _ = src_pixel_group.shape
    nt = B * S * G
    out = _sc_gather(pred.reshape(-1), src_pixel_group.reshape(-1),
                     B * N * 2, nt, H, 1.0 / slic.shape[1])
    dst = out.reshape(B, S, G, 2)
    return pred_corr, dst


# speedup vs baseline: 1.0691x; 1.0691x over previous
import functools

import jax
import jax.numpy as jnp
from jax import lax
from jax.experimental import pallas as pl
from jax.experimental.pallas import tpu as pltpu
from jax.experimental.pallas import tpu_sc as plsc

_RB = 512


def _tree_norm(x):
    t = x * x
    c = t.shape[0]
    while c > 1:
        h = c // 2
        t = t[:h] + t[h:c]
        c = h
    return jnp.sqrt(t) + 1e-12


def _corr_body(qt_ref, kt_ref, o_ref):
    n = kt_ref.shape[2]
    rb = qt_ref.shape[2]
    qt = qt_ref[0]
    kt = kt_ref[0]
    nq = _tree_norm(qt)
    nk = _tree_norm(kt)
    s0 = lax.dot_general(
        (qt / nq).astype(jnp.bfloat16), (kt / nk).astype(jnp.bfloat16),
        (((0,), (0,)), ((), ())),
        preferred_element_type=jnp.float32,
    )
    ch = 128
    neg = jnp.full((rb, ch), -jnp.inf, jnp.float32)
    v1, v2, v3 = neg, neg, neg
    for c in range(n // ch):
        x = s0[:, c * ch:(c + 1) * ch]
        t1 = jnp.maximum(v1, x)
        l1 = jnp.minimum(v1, x)
        t2 = jnp.maximum(v2, l1)
        l2 = jnp.minimum(v2, l1)
        v3 = jnp.maximum(v3, l2)
        v1, v2 = t1, t2
    width = ch
    while width > 1:
        h = width // 2
        a1, b1 = v1[:, :h], v1[:, h:width]
        a2, b2 = v2[:, :h], v2[:, h:width]
        a3, b3 = v3[:, :h], v3[:, h:width]
        u = jnp.minimum(a1, b1)
        p = jnp.maximum(a2, b2)
        v1 = jnp.maximum(a1, b1)
        v2 = jnp.maximum(u, p)
        v3 = jnp.maximum(
            jnp.maximum(jnp.minimum(u, p), jnp.minimum(a2, b2)),
            jnp.maximum(a3, b3))
        width = h
    m3 = v3
    ci = lax.broadcasted_iota(jnp.int32, (1, n), 1)
    rhs_t = jnp.concatenate(
        [nk * (ci >> 6).astype(jnp.float32),
         nk * (ci & 63).astype(jnp.float32),
         nk], axis=0)
    wm = jnp.where(s0 >= m3, s0, 0.0)
    nums = lax.dot_general(
        wm, rhs_t, (((1,), (1,)), ((), ())),
        preferred_element_type=jnp.float32,
    )
    den = nums[:, 2:3]
    o_ref[0] = jnp.concatenate([nums[:, 0:1] / den, nums[:, 1:2] / den], axis=1)


def _pred_corr(fx, fy):
    B, C, N = fx.shape
    return pl.pallas_call(
        _corr_body,
        grid=(B, N // _RB),
        in_specs=[
            pl.BlockSpec((1, C, _RB), lambda b, i: (b, 0, i)),
            pl.BlockSpec((1, C, N), lambda b, i: (b, 0, 0)),
        ],
        out_specs=pl.BlockSpec((1, _RB, 2), lambda b, i: (b, i, 0)),
        out_shape=jax.ShapeDtypeStruct((B, N, 2), jnp.float32),
    )(fx, fy)


def _sc_gather(table, idxp, n_tab, per_tile, scale):
    n_iter = per_tile // 16
    mesh = plsc.VectorSubcoreMesh(core_axis_name="c", subcore_axis_name="s")

    @functools.partial(
        pl.kernel,
        mesh=mesh,
        compiler_params=pltpu.CompilerParams(needs_layout_passes=False),
        out_type=jax.ShapeDtypeStruct((32 * 2 * per_tile,), jnp.float32),
        scratch_types=[
            pltpu.VMEM((n_tab,), jnp.float32),
            pltpu.VMEM((per_tile,), jnp.int32),
            pltpu.VMEM((2 * per_tile,), jnp.float32),
        ],
    )
    def k(table_hbm, idx_hbm, out_hbm, tab_v, idx_v, out_v):
        wid = lax.axis_index("s") * 2 + lax.axis_index("c")
        pltpu.sync_copy(table_hbm, tab_v)
        pltpu.sync_copy(idx_hbm.at[pl.ds(wid * per_tile, per_tile)], idx_v)
        lane = lax.iota(jnp.int32, 16)
        for j in range(n_iter):
            iv = idx_v[pl.ds(j * 16, 16)]
            g0 = plsc.load_gather(tab_v, [iv])
            g1 = plsc.load_gather(tab_v, [iv + 1])
            pos = j * 32 + 2 * lane
            plsc.store_scatter(out_v, [pos], g0 * scale)
            plsc.store_scatter(out_v, [pos + 1], g1 * scale)
        pltpu.sync_copy(out_v, out_hbm.at[pl.ds(wid * 2 * per_tile, 2 * per_tile)])

    return k(table, idxp)


def kernel(img1_feature, img2_feature, mask_1, mask_2, slic, src_pixel_group):
    B, C, H, W = img1_feature.shape
    N = H * W
    fy = img2_feature.reshape(B, C, N)
    fx = img1_feature.reshape(B, C, N)
    pred = _pred_corr(fx, fy)
    pred_corr = pred.reshape(B, H, W, 2)

    idx = jnp.clip(jnp.round(src_pixel_group * H).astype(jnp.int32), 0, H - 1)
    _, S, G, _ = idx.shape
    lin = (jnp.arange(B)[:, None, None] * N + idx[..., 0] * W + idx[..., 1]) * 2
    flat = lin.reshape(-1)
    nt = flat.shape[0]
    per_tile = ((nt + 31) // 32 + 15) // 16 * 16
    flat = jnp.pad(flat, (0, 32 * per_tile - nt))
    out = _sc_gather(pred.reshape(-1), flat, B * N * 2, per_tile,
                     1.0 / slic.shape[1])
    dst = out[: 2 * nt].reshape(B, S, G, 2)
    return pred_corr, dst


# speedup vs baseline: 1.1014x; 1.0302x over previous
import functools

import jax
import jax.numpy as jnp
from jax import lax
from jax.experimental import pallas as pl
from jax.experimental.pallas import tpu as pltpu
from jax.experimental.pallas import tpu_sc as plsc

_RB = 1024


def _tree_norm(x):
    t = x * x
    c = t.shape[0]
    while c > 1:
        h = c // 2
        t = t[:h] + t[h:c]
        c = h
    return jnp.sqrt(t) + 1e-12


def _corr_body(qt_ref, kt_ref, o_ref):
    n = kt_ref.shape[2]
    rb = qt_ref.shape[2]
    qt = qt_ref[0]
    kt = kt_ref[0]
    nq = _tree_norm(qt)
    nk = _tree_norm(kt)
    s0 = lax.dot_general(
        (qt / nq).astype(jnp.bfloat16), (kt / nk).astype(jnp.bfloat16),
        (((0,), (0,)), ((), ())),
        preferred_element_type=jnp.float32,
    )
    ch = 128
    neg = jnp.full((rb, ch), -jnp.inf, jnp.float32)
    v1, v2, v3 = neg, neg, neg
    for c in range(n // ch):
        x = s0[:, c * ch:(c + 1) * ch]
        t1 = jnp.maximum(v1, x)
        l1 = jnp.minimum(v1, x)
        t2 = jnp.maximum(v2, l1)
        l2 = jnp.minimum(v2, l1)
        v3 = jnp.maximum(v3, l2)
        v1, v2 = t1, t2
    width = ch
    while width > 1:
        h = width // 2
        a1, b1 = v1[:, :h], v1[:, h:width]
        a2, b2 = v2[:, :h], v2[:, h:width]
        a3, b3 = v3[:, :h], v3[:, h:width]
        u = jnp.minimum(a1, b1)
        p = jnp.maximum(a2, b2)
        v1 = jnp.maximum(a1, b1)
        v2 = jnp.maximum(u, p)
        v3 = jnp.maximum(
            jnp.maximum(jnp.minimum(u, p), jnp.minimum(a2, b2)),
            jnp.maximum(a3, b3))
        width = h
    m3 = v3
    ci = lax.broadcasted_iota(jnp.int32, (1, n), 1)
    rhs_t = jnp.concatenate(
        [nk * (ci >> 6).astype(jnp.float32),
         nk * (ci & 63).astype(jnp.float32),
         nk], axis=0)
    wm = jnp.where(s0 >= m3, s0, 0.0)
    nums = lax.dot_general(
        wm, rhs_t, (((1,), (1,)), ((), ())),
        preferred_element_type=jnp.float32,
    )
    den = nums[:, 2:3]
    o_ref[0] = jnp.concatenate([nums[:, 0:1] / den, nums[:, 1:2] / den], axis=1)


def _pred_corr(fx, fy):
    B, C, N = fx.shape
    return pl.pallas_call(
        _corr_body,
        grid=(B, N // _RB),
        in_specs=[
            pl.BlockSpec((1, C, _RB), lambda b, i: (b, 0, i)),
            pl.BlockSpec((1, C, N), lambda b, i: (b, 0, 0)),
        ],
        out_specs=pl.BlockSpec((1, _RB, 2), lambda b, i: (b, i, 0)),
        out_shape=jax.ShapeDtypeStruct((B, N, 2), jnp.float32),
    )(fx, fy)


def _sc_gather(table, idxp, n_tab, per_tile, scale):
    n_iter = per_tile // 16
    mesh = plsc.VectorSubcoreMesh(core_axis_name="c", subcore_axis_name="s")

    @functools.partial(
        pl.kernel,
        mesh=mesh,
        compiler_params=pltpu.CompilerParams(needs_layout_passes=False),
        out_type=jax.ShapeDtypeStruct((32 * 2 * per_tile,), jnp.float32),
        scratch_types=[
            pltpu.VMEM((n_tab,), jnp.float32),
            pltpu.VMEM((per_tile,), jnp.int32),
            pltpu.VMEM((2 * per_tile,), jnp.float32),
        ],
    )
    def k(table_hbm, idx_hbm, out_hbm, tab_v, idx_v, out_v):
        wid = lax.axis_index("s") * 2 + lax.axis_index("c")
        pltpu.sync_copy(table_hbm, tab_v)
        pltpu.sync_copy(idx_hbm.at[pl.ds(wid * per_tile, per_tile)], idx_v)
        lane = lax.iota(jnp.int32, 16)
        for j in range(n_iter):
            iv = idx_v[pl.ds(j * 16, 16)]
            g0 = plsc.load_gather(tab_v, [iv])
            g1 = plsc.load_gather(tab_v, [iv + 1])
            pos = j * 32 + 2 * lane
            plsc.store_scatter(out_v, [pos], g0 * scale)
            plsc.store_scatter(out_v, [pos + 1], g1 * scale)
        pltpu.sync_copy(out_v, out_hbm.at[pl.ds(wid * 2 * per_tile, 2 * per_tile)])

    return k(table, idxp)


def kernel(img1_feature, img2_feature, mask_1, mask_2, slic, src_pixel_group):
    B, C, H, W = img1_feature.shape
    N = H * W
    fy = img2_feature.reshape(B, C, N)
    fx = img1_feature.reshape(B, C, N)
    pred = _pred_corr(fx, fy)
    pred_corr = pred.reshape(B, H, W, 2)

    idx = jnp.clip(jnp.round(src_pixel_group * H).astype(jnp.int32), 0, H - 1)
    _, S, G, _ = idx.shape
    lin = (jnp.arange(B)[:, None, None] * N + idx[..., 0] * W + idx[..., 1]) * 2
    flat = lin.reshape(-1)
    nt = flat.shape[0]
    per_tile = ((nt + 31) // 32 + 15) // 16 * 16
    flat = jnp.pad(flat, (0, 32 * per_tile - nt))
    out = _sc_gather(pred.reshape(-1), flat, B * N * 2, per_tile,
                     1.0 / slic.shape[1])
    dst = out[: 2 * nt].reshape(B, S, G, 2)
    return pred_corr, dst


# speedup vs baseline: 1.1152x; 1.0125x over previous
import functools

import jax
import jax.numpy as jnp
from jax import lax
from jax.experimental import pallas as pl
from jax.experimental.pallas import tpu as pltpu
from jax.experimental.pallas import tpu_sc as plsc

_RB = 2048


def _tree_norm(x):
    t = x * x
    c = t.shape[0]
    while c > 1:
        h = c // 2
        t = t[:h] + t[h:c]
        c = h
    return jnp.sqrt(t) + 1e-12


def _corr_body(qt_ref, kt_ref, o_ref):
    n = kt_ref.shape[2]
    rb = qt_ref.shape[2]
    qt = qt_ref[0]
    kt = kt_ref[0]
    nq = _tree_norm(qt)
    nk = _tree_norm(kt)
    s0 = lax.dot_general(
        (qt / nq).astype(jnp.bfloat16), (kt / nk).astype(jnp.bfloat16),
        (((0,), (0,)), ((), ())),
        preferred_element_type=jnp.float32,
    )
    ch = 128
    neg = jnp.full((rb, ch), -jnp.inf, jnp.float32)
    v1, v2, v3 = neg, neg, neg
    for c in range(n // ch):
        x = s0[:, c * ch:(c + 1) * ch]
        t1 = jnp.maximum(v1, x)
        l1 = jnp.minimum(v1, x)
        t2 = jnp.maximum(v2, l1)
        l2 = jnp.minimum(v2, l1)
        v3 = jnp.maximum(v3, l2)
        v1, v2 = t1, t2
    width = ch
    while width > 1:
        h = width // 2
        a1, b1 = v1[:, :h], v1[:, h:width]
        a2, b2 = v2[:, :h], v2[:, h:width]
        a3, b3 = v3[:, :h], v3[:, h:width]
        u = jnp.minimum(a1, b1)
        p = jnp.maximum(a2, b2)
        v1 = jnp.maximum(a1, b1)
        v2 = jnp.maximum(u, p)
        v3 = jnp.maximum(
            jnp.maximum(jnp.minimum(u, p), jnp.minimum(a2, b2)),
            jnp.maximum(a3, b3))
        width = h
    m3 = v3
    ci = lax.broadcasted_iota(jnp.int32, (1, n), 1)
    rhs_t = jnp.concatenate(
        [nk * (ci >> 6).astype(jnp.float32),
         nk * (ci & 63).astype(jnp.float32),
         nk], axis=0)
    wm = jnp.where(s0 >= m3, s0, 0.0)
    nums = lax.dot_general(
        wm, rhs_t, (((1,), (1,)), ((), ())),
        preferred_element_type=jnp.float32,
    )
    den = nums[:, 2:3]
    o_ref[0] = jnp.concatenate([nums[:, 0:1] / den, nums[:, 1:2] / den], axis=1)


def _pred_corr(fx, fy):
    B, C, N = fx.shape
    return pl.pallas_call(
        _corr_body,
        grid=(B, N // _RB),
        in_specs=[
            pl.BlockSpec((1, C, _RB), lambda b, i: (b, 0, i)),
            pl.BlockSpec((1, C, N), lambda b, i: (b, 0, 0)),
        ],
        out_specs=pl.BlockSpec((1, _RB, 2), lambda b, i: (b, i, 0)),
        out_shape=jax.ShapeDtypeStruct((B, N, 2), jnp.float32),
    )(fx, fy)


def _sc_gather(table, idxp, n_tab, per_tile, scale):
    n_iter = per_tile // 16
    mesh = plsc.VectorSubcoreMesh(core_axis_name="c", subcore_axis_name="s")

    @functools.partial(
        pl.kernel,
        mesh=mesh,
        compiler_params=pltpu.CompilerParams(needs_layout_passes=False),
        out_type=jax.ShapeDtypeStruct((32 * 2 * per_tile,), jnp.float32),
        scratch_types=[
            pltpu.VMEM((n_tab,), jnp.float32),
            pltpu.VMEM((per_tile,), jnp.int32),
            pltpu.VMEM((2 * per_tile,), jnp.float32),
        ],
    )
    def k(table_hbm, idx_hbm, out_hbm, tab_v, idx_v, out_v):
        wid = lax.axis_index("s") * 2 + lax.axis_index("c")
        pltpu.sync_copy(table_hbm, tab_v)
        pltpu.sync_copy(idx_hbm.at[pl.ds(wid * per_tile, per_tile)], idx_v)
        lane = lax.iota(jnp.int32, 16)
        for j in range(n_iter):
            iv = idx_v[pl.ds(j * 16, 16)]
            g0 = plsc.load_gather(tab_v, [iv])
            g1 = plsc.load_gather(tab_v, [iv + 1])
            pos = j * 32 + 2 * lane
            plsc.store_scatter(out_v, [pos], g0 * scale)
            plsc.store_scatter(out_v, [pos + 1], g1 * scale)
        pltpu.sync_copy(out_v, out_hbm.at[pl.ds(wid * 2 * per_tile, 2 * per_tile)])

    return k(table, idxp)


def kernel(img1_feature, img2_feature, mask_1, mask_2, slic, src_pixel_group):
    B, C, H, W = img1_feature.shape
    N = H * W
    fy = img2_feature.reshape(B, C, N)
    fx = img1_feature.reshape(B, C, N)
    pred = _pred_corr(fx, fy)
    pred_corr = pred.reshape(B, H, W, 2)

    idx = jnp.clip(jnp.round(src_pixel_group * H).astype(jnp.int32), 0, H - 1)
    _, S, G, _ = idx.shape
    lin = (jnp.arange(B)[:, None, None] * N + idx[..., 0] * W + idx[..., 1]) * 2
    flat = lin.reshape(-1)
    nt = flat.shape[0]
    per_tile = ((nt + 31) // 32 + 15) // 16 * 16
    flat = jnp.pad(flat, (0, 32 * per_tile - nt))
    out = _sc_gather(pred.reshape(-1), flat, B * N * 2, per_tile,
                     1.0 / slic.shape[1])
    dst = out[: 2 * nt].reshape(B, S, G, 2)
    return pred_corr, dst
